# Initial kernel scaffold; baseline (speedup 1.0000x reference)
import functools

import jax
import jax.numpy as jnp
from jax import lax
from jax.experimental import pallas as pl
from jax.experimental.pallas import tpu as pltpu
from jax.experimental.pallas import tpu_sc as plsc

_N = 10000
_E = 320000
_NC = 2
_NS = 16
_NW = _NC * _NS
_CHUNK = 128
_EPW = _E // _NW
_NCH = 80
_EPWP = _NCH * _CHUNK
_PAD = _EPWP - _EPW
_NPADROW = 256
_NACC = _N + _NPADROW
_WRPT = 640
_WLAST = _N - 15 * _WRPT


def _leaky(v):
    return jnp.where(v >= 0, v, 0.01 * v)


def _bn(h2, gamma, beta):
    m = jnp.mean(h2, axis=0, keepdims=True)
    var = jnp.mean((h2 - m) * (h2 - m), axis=0, keepdims=True)
    return (h2 - m) / jnp.sqrt(var + 1e-4) * gamma + beta


def _mm(a, w):
    return lax.dot_general(a, w, (((1,), (1,)), ((), ())),
                           preferred_element_type=jnp.float32)


def _make_segsum(width):
    mesh = plsc.VectorSubcoreMesh(core_axis_name="c", subcore_axis_name="s")

    @functools.partial(
        pl.kernel,
        out_type=jax.ShapeDtypeStruct((2 * _N, width), jnp.float32),
        mesh=mesh,
        scratch_types=[
            pltpu.VMEM((_EPWP,), jnp.int32),
            pltpu.VMEM((_NCH, _CHUNK), jnp.int32),
            pltpu.VMEM((_CHUNK, width), jnp.float32),
            pltpu.VMEM_SHARED((_NACC, width), jnp.float32),
            pltpu.SemaphoreType.DMA,
        ],
    )
    def segsum(v_hbm, src_hbm, dst_hbm, zero_hbm, out_hbm,
               src_v, dst_v, rows_v, acc, sem):
        c = lax.axis_index("c")
        s = lax.axis_index("s")
        wid = s * _NC + c

        @pl.when(s < 15)
        def _():
            pltpu.sync_copy(zero_hbm, acc.at[pl.ds(s * _WRPT, _WRPT)])

        @pl.when(s == 15)
        def _():
            pltpu.sync_copy(zero_hbm.at[pl.ds(0, _WLAST)],
                            acc.at[pl.ds(15 * _WRPT, _WLAST)])

        pltpu.sync_copy(src_hbm.at[pl.ds(wid * _EPWP, _EPWP)], src_v)
        pltpu.sync_copy(dst_hbm.at[wid], dst_v)
        plsc.subcore_barrier()

        def chunk(i, carry):
            pltpu.async_copy(
                v_hbm.at[src_v.at[pl.ds(i * _CHUNK, _CHUNK)]], rows_v, sem
            ).wait()
            pltpu.sync_copy(rows_v, acc.at[dst_v.at[i]], add=True)
            return carry

        lax.fori_loop(0, _NCH, chunk, 0)
        plsc.subcore_barrier()

        @pl.when(s < 15)
        def _():
            pltpu.sync_copy(acc.at[pl.ds(s * _WRPT, _WRPT)],
                            out_hbm.at[pl.ds(c * _N + s * _WRPT, _WRPT)])

        @pl.when(s == 15)
        def _():
            pltpu.sync_copy(acc.at[pl.ds(15 * _WRPT, _WLAST)],
                            out_hbm.at[pl.ds(c * _N + 15 * _WRPT, _WLAST)])

    return segsum


_segsum64 = _make_segsum(64)
_segsum128 = _make_segsum(128)


def _tc_call(body, out_shapes, *args):
    return pl.pallas_call(body, out_shape=out_shapes)(*args)


def _tc1_body(x_ref, w1_ref, v0_ref):
    v0_ref[...] = _mm(x_ref[...], w1_ref[...])


def _tc2_body(v0_ref, s0_ref, b1_ref, w2_ref, b2_ref, g_ref, be_ref, w1n_ref,
              h1_ref, v1_ref):
    t = v0_ref[...] + s0_ref[0:_N, :] + s0_ref[_N:2 * _N, :] + b1_ref[...]
    h2 = _mm(_leaky(t), w2_ref[...]) + b2_ref[...]
    h1 = _bn(h2, g_ref[...], be_ref[...])
    h1_ref[...] = h1
    v1_ref[...] = _mm(h1, w1n_ref[...])


def _tc3_body(v1_ref, s1_ref, b1_ref, w2_ref, b2_ref, g_ref, be_ref, h1_ref,
              fw0, fb0, fw1, fb1, fw2, fb2, fws, fbs,
              hid_ref):
    t = v1_ref[...] + s1_ref[0:_N, :] + s1_ref[_N:2 * _N, :] + b1_ref[...]
    h2 = _mm(_leaky(t), w2_ref[...]) + b2_ref[...]
    hb2 = _bn(h2, g_ref[...], be_ref[...])
    hidden = jnp.concatenate([h1_ref[...], hb2], axis=1)
    b = _leaky(_mm(hidden, fw0[...]) + fb0[...])
    b = _leaky(_mm(b, fw1[...]) + fb1[...])
    b = _leaky(_mm(b, fw2[...]) + fb2[...])
    hid_ref[...] = b + _mm(hidden, fws[...]) + fbs[...]


def _tc4_body(hid_ref, s3_ref,
              mw1, mb1, mw2, mb2, mg, mbe, mw1n,
              lw1, lb1, lw2, lb2, lg, lbe, lw1n,
              hm1_ref, hl1_ref, vcat_ref):
    t = hid_ref[...] + s3_ref[0:_N, :] + s3_ref[_N:2 * _N, :]
    h2m = _mm(_leaky(_mm(t, mw1[...]) + mb1[...]), mw2[...]) + mb2[...]
    hm1 = _bn(h2m, mg[...], mbe[...])
    h2l = _mm(_leaky(_mm(t, lw1[...]) + lb1[...]), lw2[...]) + lb2[...]
    hl1 = _bn(h2l, lg[...], lbe[...])
    hm1_ref[...] = hm1
    hl1_ref[...] = hl1
    vcat_ref[...] = jnp.concatenate([_mm(hm1, mw1n[...]), _mm(hl1, lw1n[...])],
                                    axis=1)


def _ff_block(h, w0, b0, w1, b1, w2, b2, ws, bs):
    b = _leaky(_mm(h, w0) + b0)
    b = _leaky(_mm(b, w1) + b1)
    b = _leaky(_mm(b, w2) + b2)
    return b + _mm(h, ws) + bs


def _tc5_body(vcat_ref, s45_ref, hm1_ref, hl1_ref, noise_ref,
              mb1, mw2, mb2, mg, mbe,
              lb1, lw2, lb2, lg, lbe,
              mf0w, mf0b, mf1w, mf1b, mf2w, mf2b, mfsw, mfsb,
              lf0w, lf0b, lf1w, lf1b, lf2w, lf2b, lfsw, lfsb,
              z_ref):
    scat = s45_ref[0:_N, :] + s45_ref[_N:2 * _N, :]
    vcat = vcat_ref[...]
    tm = _leaky(vcat[:, 0:64] + scat[:, 0:64] + mb1[...])
    hm2 = _bn(_mm(tm, mw2[...]) + mb2[...], mg[...], mbe[...])
    tl = _leaky(vcat[:, 64:128] + scat[:, 64:128] + lb1[...])
    hl2 = _bn(_mm(tl, lw2[...]) + lb2[...], lg[...], lbe[...])
    meanc = jnp.concatenate([hm1_ref[...], hm2], axis=1)
    mean = _ff_block(meanc, mf0w[...], mf0b[...], mf1w[...], mf1b[...],
                     mf2w[...], mf2b[...], mfsw[...], mfsb[...])
    logc = jnp.concatenate([hl1_ref[...], hl2], axis=1)
    logstd = _ff_block(logc, lf0w[...], lf0b[...], lf1w[...], lf1b[...],
                       lf2w[...], lf2b[...], lfsw[...], lfsb[...])
    z_ref[...] = noise_ref[...] * jnp.exp(logstd) + mean


_BR = 2000
_BC = 2000


def _decode_body(zr_ref, zc_ref, o_ref):
    logits = lax.dot_general(zr_ref[...], zc_ref[...], (((1,), (1,)), ((), ())),
                             preferred_element_type=jnp.float32)
    o_ref[...] = jax.nn.sigmoid(logits)


def _decode(z):
    grid = (_N // _BR, _N // _BC)
    return pl.pallas_call(
        _decode_body,
        grid=grid,
        in_specs=[
            pl.BlockSpec((_BR, 128), lambda i, j: (i, 0)),
            pl.BlockSpec((_BC, 128), lambda i, j: (j, 0)),
        ],
        out_specs=pl.BlockSpec((_BR, _BC), lambda i, j: (i, j)),
        out_shape=jax.ShapeDtypeStruct((_N, _N), jnp.float32),
    )(z, z)


def _row(v):
    return v.reshape(1, -1)


def kernel(x, edge_index, batch, params, noise):
    del batch

    src = edge_index[0].reshape(_NW, _EPW)
    dst = edge_index[1].reshape(_NW, _EPW)
    pad_src = jnp.broadcast_to(
        (jnp.arange(_PAD, dtype=jnp.int32) * 37) % _N, (_NW, _PAD))
    pad_dst = jnp.broadcast_to(
        _N + (jnp.arange(_PAD, dtype=jnp.int32) % _NPADROW), (_NW, _PAD))
    src_p = jnp.concatenate([src, pad_src], axis=1).reshape(-1)
    dst_p = jnp.concatenate([dst, pad_dst], axis=1).reshape(_NW, _NCH, _CHUNK)
    zeros64 = jnp.zeros((_WRPT, 64), jnp.float32)
    zeros128 = jnp.zeros((_WRPT, 128), jnp.float32)

    p_base0, p_base1 = params['base']
    p_m0, p_m1 = params['mean']
    p_l0, p_l1 = params['logstd']
    ffb, ffm, ffl = params['ff_base'], params['ff_mean'], params['ff_logstd']

    v0 = _tc_call(_tc1_body, jax.ShapeDtypeStruct((_N, 64), jnp.float32),
                  x, p_base0['W1'])
    s0 = _segsum64(v0, src_p, dst_p, zeros64)
    h1, v1 = _tc_call(
        _tc2_body,
        [jax.ShapeDtypeStruct((_N, 64), jnp.float32),
         jax.ShapeDtypeStruct((_N, 64), jnp.float32)],
        v0, s0, _row(p_base0['b1']), p_base0['W2'], _row(p_base0['b2']),
        _row(p_base0['gamma']), _row(p_base0['beta']), p_base1['W1'])

    s1 = _segsum64(v1, src_p, dst_p, zeros64)
    (fW0, fb0), (fW1, fb1), (fW2, fb2) = ffb['L']
    fWs, fbs = ffb['S']
    hidden = _tc_call(
        _tc3_body, jax.ShapeDtypeStruct((_N, 128), jnp.float32),
        v1, s1, _row(p_base1['b1']), p_base1['W2'], _row(p_base1['b2']),
        _row(p_base1['gamma']), _row(p_base1['beta']), h1,
        fW0, _row(fb0), fW1, _row(fb1), fW2, _row(fb2), fWs, _row(fbs))

    s3 = _segsum128(hidden, src_p, dst_p, zeros128)
    hm1, hl1, vcat = _tc_call(
        _tc4_body,
        [jax.ShapeDtypeStruct((_N, 64), jnp.float32),
         jax.ShapeDtypeStruct((_N, 64), jnp.float32),
         jax.ShapeDtypeStruct((_N, 128), jnp.float32)],
        hidden, s3,
        p_m0['W1'], _row(p_m0['b1']), p_m0['W2'], _row(p_m0['b2']),
        _row(p_m0['gamma']), _row(p_m0['beta']), p_m1['W1'],
        p_l0['W1'], _row(p_l0['b1']), p_l0['W2'], _row(p_l0['b2']),
        _row(p_l0['gamma']), _row(p_l0['beta']), p_l1['W1'])

    s45 = _segsum128(vcat, src_p, dst_p, zeros128)
    (mf0w, mf0b), (mf1w, mf1b), (mf2w, mf2b) = ffm['L']
    mfsw, mfsb = ffm['S']
    (lf0w, lf0b), (lf1w, lf1b), (lf2w, lf2b) = ffl['L']
    lfsw, lfsb = ffl['S']
    z = _tc_call(
        _tc5_body, jax.ShapeDtypeStruct((_N, 128), jnp.float32),
        vcat, s45, hm1, hl1, noise,
        _row(p_m1['b1']), p_m1['W2'], _row(p_m1['b2']),
        _row(p_m1['gamma']), _row(p_m1['beta']),
        _row(p_l1['b1']), p_l1['W2'], _row(p_l1['b2']),
        _row(p_l1['gamma']), _row(p_l1['beta']),
        mf0w, _row(mf0b), mf1w, _row(mf1b), mf2w, _row(mf2b), mfsw, _row(mfsb),
        lf0w, _row(lf0b), lf1w, _row(lf1b), lf2w, _row(lf2b), lfsw, _row(lfsb))

    return _decode(z)


# speedup vs baseline: 4.7422x; 4.7422x over previous
import functools

import jax
import jax.numpy as jnp
from jax import lax
from jax.experimental import pallas as pl
from jax.experimental.pallas import tpu as pltpu
from jax.experimental.pallas import tpu_sc as plsc

_N = 10000
_E = 320000
_NC = 2
_NS = 16
_NW = _NC * _NS
_CHUNK = 128
_EPW = _E // _NW
_NCH = 80
_EPWP = _NCH * _CHUNK
_PAD = _EPWP - _EPW
_NPADROW = 256
_NACC = _N + _NPADROW
_WRPT = 640
_WLAST = _N - 15 * _WRPT


def _leaky(v):
    return jnp.where(v >= 0, v, 0.01 * v)


def _tree_colsum(a):
    parts = [jnp.sum(a[i * 625:(i + 1) * 625, :], axis=0, keepdims=True)
             for i in range(16)]
    while len(parts) > 1:
        parts = [parts[i] + parts[i + 1] for i in range(0, len(parts), 2)]
    return parts[0]


def _bn(h2, gamma, beta):
    m = _tree_colsum(h2) * (1.0 / _N)
    d = h2 - m
    var = _tree_colsum(d * d) * (1.0 / _N)
    return d / jnp.sqrt(var + 1e-4) * gamma + beta


def _mm(a, w):
    return lax.dot_general(a.astype(jnp.bfloat16), w.astype(jnp.bfloat16),
                           (((1,), (1,)), ((), ())),
                           preferred_element_type=jnp.float32)


def _make_segsum(width):
    mesh = plsc.VectorSubcoreMesh(core_axis_name="c", subcore_axis_name="s",
                                  num_cores=_NC, num_subcores=_NS)

    @functools.partial(
        pl.kernel,
        out_type=jax.ShapeDtypeStruct((2 * _N, width), jnp.float32),
        mesh=mesh,
        scratch_types=[
            pltpu.VMEM((_EPWP,), jnp.int32),
            pltpu.VMEM((_NCH, _CHUNK), jnp.int32),
            pltpu.VMEM((_CHUNK, width), jnp.float32),
            pltpu.VMEM_SHARED((_NACC, width), jnp.float32),
            pltpu.SemaphoreType.DMA,
        ],
    )
    def segsum(v_hbm, src_hbm, dst_hbm, zero_hbm, out_hbm,
               src_v, dst_v, rows_v, acc, sem):
        c = lax.axis_index("c")
        s = lax.axis_index("s")
        wid = s * _NC + c

        @pl.when(s < 15)
        def _():
            pltpu.sync_copy(zero_hbm, acc.at[pl.ds(s * _WRPT, _WRPT)])

        @pl.when(s == 15)
        def _():
            pltpu.sync_copy(zero_hbm.at[pl.ds(0, _WLAST)],
                            acc.at[pl.ds(15 * _WRPT, _WLAST)])

        pltpu.sync_copy(src_hbm.at[pl.ds(wid * _EPWP, _EPWP)], src_v)
        pltpu.sync_copy(dst_hbm.at[wid], dst_v)
        plsc.subcore_barrier()

        def chunk(i, carry):
            pltpu.async_copy(
                v_hbm.at[src_v.at[pl.ds(i * _CHUNK, _CHUNK)]], rows_v, sem
            ).wait()
            pltpu.sync_copy(rows_v, acc.at[dst_v.at[i]], add=True)
            return carry

        lax.fori_loop(0, _NCH, chunk, 0)
        plsc.subcore_barrier()

        @pl.when(s < 15)
        def _():
            pltpu.sync_copy(acc.at[pl.ds(s * _WRPT, _WRPT)],
                            out_hbm.at[pl.ds(c * _N + s * _WRPT, _WRPT)])

        @pl.when(s == 15)
        def _():
            pltpu.sync_copy(acc.at[pl.ds(15 * _WRPT, _WLAST)],
                            out_hbm.at[pl.ds(c * _N + 15 * _WRPT, _WLAST)])

    return segsum


@functools.lru_cache(maxsize=None)
def _get_segsum(width):
    return _make_segsum(width)


def _segsum(width, *args):
    return _get_segsum(width)(*args)


def _tc_call(body, out_shapes, *args):
    return pl.pallas_call(body, out_shape=out_shapes)(*args)


def _tc2_body(x_ref, s0_ref, w1_ref, b1_ref, w2_ref, b2_ref, g_ref, be_ref,
              h1z_ref):
    t = x_ref[...] + s0_ref[0:_N, :] + s0_ref[_N:2 * _N, :]
    h2 = _mm(_leaky(_mm(t, w1_ref[...]) + b1_ref[...]), w2_ref[...]) + b2_ref[...]
    h1 = _bn(h2, g_ref[...], be_ref[...])
    h1z_ref[...] = jnp.concatenate([h1, jnp.zeros_like(h1)], axis=1)


def _tc3_body(h1z_ref, s1_ref, w1_ref, b1_ref, w2_ref, b2_ref, g_ref, be_ref,
              fw0, fb0, fw1, fb1, fw2, fb2, fws, fbs,
              hid_ref):
    t = (h1z_ref[:, 0:64] + s1_ref[0:_N, 0:64] + s1_ref[_N:2 * _N, 0:64])
    h2 = _mm(_leaky(_mm(t, w1_ref[...]) + b1_ref[...]), w2_ref[...]) + b2_ref[...]
    hb2 = _bn(h2, g_ref[...], be_ref[...])
    hidden = jnp.concatenate([h1z_ref[:, 0:64], hb2], axis=1)
    b = _leaky(_mm(hidden, fw0[...]) + fb0[...])
    b = _leaky(_mm(b, fw1[...]) + fb1[...])
    b = _leaky(_mm(b, fw2[...]) + fb2[...])
    hid_ref[...] = b + _mm(hidden, fws[...]) + fbs[...]


def _tc4_body(hid_ref, s3_ref,
              mw1, mb1, mw2, mb2, mg, mbe,
              lw1, lb1, lw2, lb2, lg, lbe,
              hcat_ref):
    t = hid_ref[...] + s3_ref[0:_N, :] + s3_ref[_N:2 * _N, :]
    h2m = _mm(_leaky(_mm(t, mw1[...]) + mb1[...]), mw2[...]) + mb2[...]
    hm1 = _bn(h2m, mg[...], mbe[...])
    h2l = _mm(_leaky(_mm(t, lw1[...]) + lb1[...]), lw2[...]) + lb2[...]
    hl1 = _bn(h2l, lg[...], lbe[...])
    hcat_ref[...] = jnp.concatenate([hm1, hl1], axis=1)


def _ff_block(h, w0, b0, w1, b1, w2, b2, ws, bs):
    b = _leaky(_mm(h, w0) + b0)
    b = _leaky(_mm(b, w1) + b1)
    b = _leaky(_mm(b, w2) + b2)
    return b + _mm(h, ws) + bs


def _tc5_body(hcat_ref, s45_ref, noise_ref,
              mw1, mb1, mw2, mb2, mg, mbe,
              lw1, lb1, lw2, lb2, lg, lbe,
              mf0w, mf0b, mf1w, mf1b, mf2w, mf2b, mfsw, mfsb,
              lf0w, lf0b, lf1w, lf1b, lf2w, lf2b, lfsw, lfsb,
              z_ref):
    scat = s45_ref[0:_N, :] + s45_ref[_N:2 * _N, :]
    hcat = hcat_ref[...]
    tm = hcat[:, 0:64] + scat[:, 0:64]
    h2m = _mm(_leaky(_mm(tm, mw1[...]) + mb1[...]), mw2[...]) + mb2[...]
    hm2 = _bn(h2m, mg[...], mbe[...])
    tl = hcat[:, 64:128] + scat[:, 64:128]
    h2l = _mm(_leaky(_mm(tl, lw1[...]) + lb1[...]), lw2[...]) + lb2[...]
    hl2 = _bn(h2l, lg[...], lbe[...])
    meanc = jnp.concatenate([hcat[:, 0:64], hm2], axis=1)
    mean = _ff_block(meanc, mf0w[...], mf0b[...], mf1w[...], mf1b[...],
                     mf2w[...], mf2b[...], mfsw[...], mfsb[...])
    logc = jnp.concatenate([hcat[:, 64:128], hl2], axis=1)
    logstd = _ff_block(logc, lf0w[...], lf0b[...], lf1w[...], lf1b[...],
                       lf2w[...], lf2b[...], lfsw[...], lfsb[...])
    z_ref[...] = noise_ref[...] * jnp.exp(logstd) + mean


_BR = 2000
_BC = 2048


def _decode_body(zr_ref, zc_ref, o_ref):
    logits = lax.dot_general(zr_ref[...].astype(jnp.bfloat16),
                             zc_ref[...].astype(jnp.bfloat16),
                             (((1,), (1,)), ((), ())),
                             preferred_element_type=jnp.float32)
    o_ref[...] = jax.nn.sigmoid(logits)


def _decode(z):
    grid = (_N // _BR, (_N + _BC - 1) // _BC)
    return pl.pallas_call(
        _decode_body,
        grid=grid,
        in_specs=[
            pl.BlockSpec((_BR, 128), lambda i, j: (i, 0)),
            pl.BlockSpec((_BC, 128), lambda i, j: (j, 0)),
        ],
        out_specs=pl.BlockSpec((_BR, _BC), lambda i, j: (i, j)),
        out_shape=jax.ShapeDtypeStruct((_N, _N), jnp.float32),
    )(z, z)


def _row(v):
    return v.reshape(1, -1)


def kernel(x, edge_index, batch, params, noise):
    del batch

    order = jnp.argsort(edge_index[1], stable=True)
    src = edge_index[0][order].reshape(_NW, _EPW)
    dst = edge_index[1][order].reshape(_NW, _EPW)
    pad_src = jnp.broadcast_to(
        (jnp.arange(_PAD, dtype=jnp.int32) * 37) % _N, (_NW, _PAD))
    pad_dst = jnp.broadcast_to(
        _N + (jnp.arange(_PAD, dtype=jnp.int32) % _NPADROW), (_NW, _PAD))
    src_p = jnp.concatenate([src, pad_src], axis=1).reshape(-1)
    dst_p = jnp.concatenate([dst, pad_dst], axis=1).reshape(_NW, _NCH, _CHUNK)
    zeros128 = jnp.zeros((_WRPT, 128), jnp.float32)

    p_base0, p_base1 = params['base']
    p_m0, p_m1 = params['mean']
    p_l0, p_l1 = params['logstd']
    ffb, ffm, ffl = params['ff_base'], params['ff_mean'], params['ff_logstd']

    s0 = _segsum(128, x, src_p, dst_p, zeros128)
    h1z = _tc_call(
        _tc2_body, jax.ShapeDtypeStruct((_N, 128), jnp.float32),
        x, s0, p_base0['W1'], _row(p_base0['b1']), p_base0['W2'],
        _row(p_base0['b2']), _row(p_base0['gamma']), _row(p_base0['beta']))

    s1 = _segsum(128, h1z, src_p, dst_p, zeros128)
    (fW0, fb0), (fW1, fb1), (fW2, fb2) = ffb['L']
    fWs, fbs = ffb['S']
    hidden = _tc_call(
        _tc3_body, jax.ShapeDtypeStruct((_N, 128), jnp.float32),
        h1z, s1, p_base1['W1'], _row(p_base1['b1']), p_base1['W2'],
        _row(p_base1['b2']), _row(p_base1['gamma']), _row(p_base1['beta']),
        fW0, _row(fb0), fW1, _row(fb1), fW2, _row(fb2), fWs, _row(fbs))

    s3 = _segsum(128, hidden, src_p, dst_p, zeros128)
    hcat = _tc_call(
        _tc4_body, jax.ShapeDtypeStruct((_N, 128), jnp.float32),
        hidden, s3,
        p_m0['W1'], _row(p_m0['b1']), p_m0['W2'], _row(p_m0['b2']),
        _row(p_m0['gamma']), _row(p_m0['beta']),
        p_l0['W1'], _row(p_l0['b1']), p_l0['W2'], _row(p_l0['b2']),
        _row(p_l0['gamma']), _row(p_l0['beta']))

    s45 = _segsum(128, hcat, src_p, dst_p, zeros128)
    (mf0w, mf0b), (mf1w, mf1b), (mf2w, mf2b) = ffm['L']
    mfsw, mfsb = ffm['S']
    (lf0w, lf0b), (lf1w, lf1b), (lf2w, lf2b) = ffl['L']
    lfsw, lfsb = ffl['S']
    z = _tc_call(
        _tc5_body, jax.ShapeDtypeStruct((_N, 128), jnp.float32),
        hcat, s45, noise,
        p_m1['W1'], _row(p_m1['b1']), p_m1['W2'], _row(p_m1['b2']),
        _row(p_m1['gamma']), _row(p_m1['beta']),
        p_l1['W1'], _row(p_l1['b1']), p_l1['W2'], _row(p_l1['b2']),
        _row(p_l1['gamma']), _row(p_l1['beta']),
        mf0w, _row(mf0b), mf1w, _row(mf1b), mf2w, _row(mf2b), mfsw, _row(mfsb),
        lf0w, _row(lf0b), lf1w, _row(lf1b), lf2w, _row(lf2b), lfsw, _row(lfsb))

    return _decode(z)


# speedup vs baseline: 5.3614x; 1.1306x over previous
import functools

import jax
import jax.numpy as jnp
from jax import lax
from jax.experimental import pallas as pl
from jax.experimental.pallas import tpu as pltpu
from jax.experimental.pallas import tpu_sc as plsc

_N = 10000
_E = 320000
_NC = 2
_NS = 16
_NW = _NC * _NS
_CHUNK = 96
_EPW = _E // _NW
_NCH = 106
_EPWP = _NCH * _CHUNK
_PAD = _EPWP - _EPW
_NPADROW = 128
_NACC = _N + _NPADROW
_WRPT = 640
_WLAST = _N - 15 * _WRPT


def _leaky(v):
    return jnp.where(v >= 0, v, 0.01 * v)


def _bn(h2, gamma, beta):
    m = jnp.mean(h2, axis=0, keepdims=True)
    var = jnp.mean((h2 - m) * (h2 - m), axis=0, keepdims=True)
    return (h2 - m) / jnp.sqrt(var + 1e-4) * gamma + beta


def _mm(a, w):
    return lax.dot_general(a.astype(jnp.bfloat16), w.astype(jnp.bfloat16),
                           (((1,), (1,)), ((), ())),
                           preferred_element_type=jnp.float32)


def _make_segsum(width):
    mesh = plsc.VectorSubcoreMesh(core_axis_name="c", subcore_axis_name="s",
                                  num_cores=_NC, num_subcores=_NS)

    @functools.partial(
        pl.kernel,
        out_type=jax.ShapeDtypeStruct((2 * _N, width), jnp.float32),
        mesh=mesh,
        scratch_types=[
            pltpu.VMEM((_EPWP,), jnp.int32),
            pltpu.VMEM((_NCH, _CHUNK), jnp.int32),
            pltpu.VMEM((_CHUNK, width), jnp.float32),
            pltpu.VMEM((_CHUNK, width), jnp.float32),
            pltpu.VMEM_SHARED((_NACC, width), jnp.float32),
            pltpu.SemaphoreType.DMA,
            pltpu.SemaphoreType.DMA,
        ],
    )
    def segsum(v_hbm, src_hbm, dst_hbm, zero_hbm, out_hbm,
               src_v, dst_v, rows0, rows1, acc, sem0, sem1):
        c = lax.axis_index("c")
        s = lax.axis_index("s")
        wid = s * _NC + c

        @pl.when(s < 15)
        def _():
            pltpu.sync_copy(zero_hbm, acc.at[pl.ds(s * _WRPT, _WRPT)])

        @pl.when(s == 15)
        def _():
            pltpu.sync_copy(zero_hbm.at[pl.ds(0, _WLAST)],
                            acc.at[pl.ds(15 * _WRPT, _WLAST)])

        pltpu.sync_copy(src_hbm.at[pl.ds(wid * _EPWP, _EPWP)], src_v)
        pltpu.sync_copy(dst_hbm.at[wid], dst_v)
        plsc.subcore_barrier()

        def gather(i, buf, sem):
            pltpu.async_copy(
                v_hbm.at[src_v.at[pl.ds(i * _CHUNK, _CHUNK)]], buf, sem)

        def wait_gather(i, buf, sem):
            pltpu.make_async_copy(
                v_hbm.at[src_v.at[pl.ds(i * _CHUNK, _CHUNK)]], buf, sem).wait()

        gather(0, rows0, sem0)

        def step(j, carry):
            i0 = 2 * j
            gather(i0 + 1, rows1, sem1)
            wait_gather(i0, rows0, sem0)
            pltpu.sync_copy(rows0, acc.at[dst_v.at[i0]], add=True)

            @pl.when(j < _NCH // 2 - 1)
            def _():
                gather(i0 + 2, rows0, sem0)

            wait_gather(i0 + 1, rows1, sem1)
            pltpu.sync_copy(rows1, acc.at[dst_v.at[i0 + 1]], add=True)
            return carry

        lax.fori_loop(0, _NCH // 2, step, 0)
        plsc.subcore_barrier()

        @pl.when(s < 15)
        def _():
            pltpu.sync_copy(acc.at[pl.ds(s * _WRPT, _WRPT)],
                            out_hbm.at[pl.ds(c * _N + s * _WRPT, _WRPT)])

        @pl.when(s == 15)
        def _():
            pltpu.sync_copy(acc.at[pl.ds(15 * _WRPT, _WLAST)],
                            out_hbm.at[pl.ds(c * _N + 15 * _WRPT, _WLAST)])

    return segsum


@functools.lru_cache(maxsize=None)
def _get_segsum(width):
    return _make_segsum(width)


def _segsum(width, *args):
    return _get_segsum(width)(*args)


def _tc_call(body, out_shapes, *args):
    return pl.pallas_call(body, out_shape=out_shapes)(*args)


def _tc2_body(x_ref, s0_ref, w1_ref, b1_ref, w2_ref, b2_ref, g_ref, be_ref,
              h1z_ref):
    t = x_ref[...] + s0_ref[0:_N, :] + s0_ref[_N:2 * _N, :]
    h2 = _mm(_leaky(_mm(t, w1_ref[...]) + b1_ref[...]), w2_ref[...]) + b2_ref[...]
    h1 = _bn(h2, g_ref[...], be_ref[...])
    h1z_ref[...] = jnp.concatenate([h1, jnp.zeros_like(h1)], axis=1)


def _tc3_body(h1z_ref, s1_ref, w1_ref, b1_ref, w2_ref, b2_ref, g_ref, be_ref,
              fw0, fb0, fw1, fb1, fw2, fb2, fws, fbs,
              hid_ref):
    t = (h1z_ref[:, 0:64] + s1_ref[0:_N, 0:64] + s1_ref[_N:2 * _N, 0:64])
    h2 = _mm(_leaky(_mm(t, w1_ref[...]) + b1_ref[...]), w2_ref[...]) + b2_ref[...]
    hb2 = _bn(h2, g_ref[...], be_ref[...])
    hidden = jnp.concatenate([h1z_ref[:, 0:64], hb2], axis=1)
    b = _leaky(_mm(hidden, fw0[...]) + fb0[...])
    b = _leaky(_mm(b, fw1[...]) + fb1[...])
    b = _leaky(_mm(b, fw2[...]) + fb2[...])
    hid_ref[...] = b + _mm(hidden, fws[...]) + fbs[...]


def _tc4_body(hid_ref, s3_ref,
              mw1, mb1, mw2, mb2, mg, mbe,
              lw1, lb1, lw2, lb2, lg, lbe,
              hcat_ref):
    t = hid_ref[...] + s3_ref[0:_N, :] + s3_ref[_N:2 * _N, :]
    h2m = _mm(_leaky(_mm(t, mw1[...]) + mb1[...]), mw2[...]) + mb2[...]
    hm1 = _bn(h2m, mg[...], mbe[...])
    h2l = _mm(_leaky(_mm(t, lw1[...]) + lb1[...]), lw2[...]) + lb2[...]
    hl1 = _bn(h2l, lg[...], lbe[...])
    hcat_ref[...] = jnp.concatenate([hm1, hl1], axis=1)


def _ff_block(h, w0, b0, w1, b1, w2, b2, ws, bs):
    b = _leaky(_mm(h, w0) + b0)
    b = _leaky(_mm(b, w1) + b1)
    b = _leaky(_mm(b, w2) + b2)
    return b + _mm(h, ws) + bs


def _tc5_body(hcat_ref, s45_ref, noise_ref,
              mw1, mb1, mw2, mb2, mg, mbe,
              lw1, lb1, lw2, lb2, lg, lbe,
              mf0w, mf0b, mf1w, mf1b, mf2w, mf2b, mfsw, mfsb,
              lf0w, lf0b, lf1w, lf1b, lf2w, lf2b, lfsw, lfsb,
              z_ref):
    scat = s45_ref[0:_N, :] + s45_ref[_N:2 * _N, :]
    hcat = hcat_ref[...]
    tm = hcat[:, 0:64] + scat[:, 0:64]
    h2m = _mm(_leaky(_mm(tm, mw1[...]) + mb1[...]), mw2[...]) + mb2[...]
    hm2 = _bn(h2m, mg[...], mbe[...])
    tl = hcat[:, 64:128] + scat[:, 64:128]
    h2l = _mm(_leaky(_mm(tl, lw1[...]) + lb1[...]), lw2[...]) + lb2[...]
    hl2 = _bn(h2l, lg[...], lbe[...])
    meanc = jnp.concatenate([hcat[:, 0:64], hm2], axis=1)
    mean = _ff_block(meanc, mf0w[...], mf0b[...], mf1w[...], mf1b[...],
                     mf2w[...], mf2b[...], mfsw[...], mfsb[...])
    logc = jnp.concatenate([hcat[:, 64:128], hl2], axis=1)
    logstd = _ff_block(logc, lf0w[...], lf0b[...], lf1w[...], lf1b[...],
                       lf2w[...], lf2b[...], lfsw[...], lfsb[...])
    z_ref[...] = noise_ref[...] * jnp.exp(logstd) + mean


_BR = 2000
_BC = 2048


def _decode_body(zr_ref, zc_ref, o_ref):
    logits = lax.dot_general(zr_ref[...].astype(jnp.bfloat16),
                             zc_ref[...].astype(jnp.bfloat16),
                             (((1,), (1,)), ((), ())),
                             preferred_element_type=jnp.float32)
    o_ref[...] = jax.nn.sigmoid(logits)


def _decode(z):
    grid = (_N // _BR, (_N + _BC - 1) // _BC)
    return pl.pallas_call(
        _decode_body,
        grid=grid,
        in_specs=[
            pl.BlockSpec((_BR, 128), lambda i, j: (i, 0)),
            pl.BlockSpec((_BC, 128), lambda i, j: (j, 0)),
        ],
        out_specs=pl.BlockSpec((_BR, _BC), lambda i, j: (i, j)),
        out_shape=jax.ShapeDtypeStruct((_N, _N), jnp.float32),
    )(z, z)


def _row(v):
    return v.reshape(1, -1)


def kernel(x, edge_index, batch, params, noise):
    del batch

    order = jnp.argsort(edge_index[1], stable=True)
    src_s = edge_index[0][order]
    dst_s = edge_index[1][order]
    cand = jnp.arange(1, _NW, dtype=jnp.int32) * _EPW
    run_end = jnp.searchsorted(dst_s, dst_s[cand], side='right')
    starts = jnp.concatenate([
        jnp.zeros((1,), jnp.int32),
        jnp.minimum(run_end, cand + _PAD).astype(jnp.int32)])
    lens = jnp.concatenate([starts[1:], jnp.full((1,), _E, jnp.int32)]) - starts
    pos = jnp.arange(_EPWP, dtype=jnp.int32)
    idx = jnp.minimum(starts[:, None] + pos[None, :], _E - 1)
    valid = pos[None, :] < lens[:, None]
    pad_src_row = (pos * 37) % _N
    pad_dst_row = _N + (pos % _NPADROW)
    src_p = jnp.where(valid, src_s[idx], pad_src_row[None, :]).reshape(-1)
    dst_p = jnp.where(valid, dst_s[idx],
                      pad_dst_row[None, :]).reshape(_NW, _NCH, _CHUNK)
    zeros128 = jnp.zeros((_WRPT, 128), jnp.float32)

    p_base0, p_base1 = params['base']
    p_m0, p_m1 = params['mean']
    p_l0, p_l1 = params['logstd']
    ffb, ffm, ffl = params['ff_base'], params['ff_mean'], params['ff_logstd']

    s0 = _segsum(128, x, src_p, dst_p, zeros128)
    h1z = _tc_call(
        _tc2_body, jax.ShapeDtypeStruct((_N, 128), jnp.float32),
        x, s0, p_base0['W1'], _row(p_base0['b1']), p_base0['W2'],
        _row(p_base0['b2']), _row(p_base0['gamma']), _row(p_base0['beta']))

    s1 = _segsum(128, h1z, src_p, dst_p, zeros128)
    (fW0, fb0), (fW1, fb1), (fW2, fb2) = ffb['L']
    fWs, fbs = ffb['S']
    hidden = _tc_call(
        _tc3_body, jax.ShapeDtypeStruct((_N, 128), jnp.float32),
        h1z, s1, p_base1['W1'], _row(p_base1['b1']), p_base1['W2'],
        _row(p_base1['b2']), _row(p_base1['gamma']), _row(p_base1['beta']),
        fW0, _row(fb0), fW1, _row(fb1), fW2, _row(fb2), fWs, _row(fbs))

    s3 = _segsum(128, hidden, src_p, dst_p, zeros128)
    hcat = _tc_call(
        _tc4_body, jax.ShapeDtypeStruct((_N, 128), jnp.float32),
        hidden, s3,
        p_m0['W1'], _row(p_m0['b1']), p_m0['W2'], _row(p_m0['b2']),
        _row(p_m0['gamma']), _row(p_m0['beta']),
        p_l0['W1'], _row(p_l0['b1']), p_l0['W2'], _row(p_l0['b2']),
        _row(p_l0['gamma']), _row(p_l0['beta']))

    s45 = _segsum(128, hcat, src_p, dst_p, zeros128)
    (mf0w, mf0b), (mf1w, mf1b), (mf2w, mf2b) = ffm['L']
    mfsw, mfsb = ffm['S']
    (lf0w, lf0b), (lf1w, lf1b), (lf2w, lf2b) = ffl['L']
    lfsw, lfsb = ffl['S']
    z = _tc_call(
        _tc5_body, jax.ShapeDtypeStruct((_N, 128), jnp.float32),
        hcat, s45, noise,
        p_m1['W1'], _row(p_m1['b1']), p_m1['W2'], _row(p_m1['b2']),
        _row(p_m1['gamma']), _row(p_m1['beta']),
        p_l1['W1'], _row(p_l1['b1']), p_l1['W2'], _row(p_l1['b2']),
        _row(p_l1['gamma']), _row(p_l1['beta']),
        mf0w, _row(mf0b), mf1w, _row(mf1b), mf2w, _row(mf2b), mfsw, _row(mfsb),
        lf0w, _row(lf0b), lf1w, _row(lf1b), lf2w, _row(lf2b), lfsw, _row(lfsb))

    return _decode(z)
